# baseline (speedup 1.0000x reference)
import functools

import jax
import jax.numpy as jnp
from jax import lax
from jax.experimental import pallas as pl
from jax.experimental.pallas import tpu as pltpu
from jax.experimental.pallas import tpu_sc as plsc

_EPS = 1e-08


def _softmax_body(x_ref, w_ref, p_ref):
    logits = jnp.dot(x_ref[...], w_ref[...], preferred_element_type=jnp.float32)
    m = jnp.max(logits, axis=1, keepdims=True)
    e = jnp.exp(logits - m)
    p_ref[...] = e / jnp.sum(e, axis=1, keepdims=True)


def _make_edge_kernel(n, e_total, d, k, nc, ns, block):
    nw = nc * ns
    epw = e_total // nw
    nit = epw // block
    assert e_total % nw == 0 and epw % block == 0
    assert block % 16 == 0 and block % 8 == 0 and block <= 128

    mesh = plsc.VectorSubcoreMesh(core_axis_name="c", subcore_axis_name="s")

    @functools.partial(
        pl.kernel,
        mesh=mesh,
        compiler_params=pltpu.CompilerParams(
            needs_layout_passes=False, use_tc_tiling_on_sc=False),
        out_type=jax.ShapeDtypeStruct((nw, 2, 16, 16), jnp.float32),
        scratch_types=[
            pltpu.VMEM((epw,), jnp.int32),
            pltpu.VMEM((epw,), jnp.int32),
            pltpu.VMEM((block, d // 2), jnp.int32),
            pltpu.VMEM((block, d // 2), jnp.int32),
            pltpu.VMEM((block, d // 2), jnp.int32),
            pltpu.VMEM((block, d // 2), jnp.int32),
            pltpu.VMEM((block, k), jnp.float32),
            pltpu.VMEM((block, k), jnp.float32),
            pltpu.VMEM((block, k), jnp.float32),
            pltpu.VMEM((block, k), jnp.float32),
            pltpu.VMEM((k, 16), jnp.float32),
            pltpu.VMEM((k, 16), jnp.float32),
            pltpu.SemaphoreType.DMA,
            pltpu.SemaphoreType.DMA,
        ],
    )
    def edge_kernel(src_hbm, tgt_hbm, x_hbm, p_hbm, out_hbm,
                    sidx_all, tidx_all, srow0, srow1, trow0, trow1,
                    sp0, sp1, tp0, tp1, a_v, b_v, sem0, sem1):
        wid = lax.axis_index("s") * nc + lax.axis_index("c")
        zero16 = jnp.zeros((16,), jnp.float32)
        for kk in range(k):
            a_v[kk, :] = zero16
            b_v[kk, :] = zero16

        lanes = lax.iota(jnp.int32, 16)
        bufs = ((srow0, trow0, sp0, tp0, sem0), (srow1, trow1, sp1, tp1, sem1))

        def issue(c, bi):
            srow, trow, sp, tp, sem = bufs[bi]
            si = sidx_all.at[pl.ds(c * block, block)]
            ti = tidx_all.at[pl.ds(c * block, block)]
            pltpu.async_copy(x_hbm.at[si], srow, sem)
            pltpu.async_copy(x_hbm.at[ti], trow, sem)
            pltpu.async_copy(p_hbm.at[si], sp, sem)
            pltpu.async_copy(p_hbm.at[ti], tp, sem)

        def drain(bi):
            srow, trow, sp, tp, sem = bufs[bi]
            pltpu.make_async_copy(x_hbm.at[pl.ds(0, block)], srow, sem).wait()
            pltpu.make_async_copy(x_hbm.at[pl.ds(0, block)], trow, sem).wait()
            pltpu.make_async_copy(p_hbm.at[pl.ds(0, block)], sp, sem).wait()
            pltpu.make_async_copy(p_hbm.at[pl.ds(0, block)], tp, sem).wait()

        dp = d // 2
        swz = (lanes * 17) & (dp - 1)

        def compute(bi):
            srow, trow, sp, tp, _ = bufs[bi]
            for g in range(block // 16):
                rows = lanes + (g * 16)

                def dstep(j, acc):
                    col = (swz + j) & (dp - 1)
                    s = plsc.load_gather(srow, [rows, col])
                    t = plsc.load_gather(trow, [rows, col])
                    s_lo = plsc.bitcast(lax.shift_left(s, 16), jnp.float32)
                    t_lo = plsc.bitcast(lax.shift_left(t, 16), jnp.float32)
                    s_hi = plsc.bitcast(s, jnp.float32)
                    t_hi = plsc.bitcast(t, jnp.float32)
                    d0 = s_lo - t_lo
                    d1 = s_hi - t_hi
                    return acc + (d0 * d0 + d1 * d1)

                dist = lax.fori_loop(0, dp, dstep, jnp.zeros((16,), jnp.float32),
                                     unroll=8)
                w = jnp.exp(dist * -0.5)
                for kk in range(k):
                    col = (lanes + kk) & (k - 1)
                    ps = plsc.load_gather(sp, [rows, col])
                    pt = plsc.load_gather(tp, [rows, col])
                    wp = w * ps
                    plsc.addupdate_scatter(a_v, [col, lanes], wp)
                    plsc.addupdate_scatter(b_v, [col, lanes], wp * pt)

        pltpu.sync_copy(src_hbm.at[pl.ds(wid * epw, epw)], sidx_all)
        pltpu.sync_copy(tgt_hbm.at[pl.ds(wid * epw, epw)], tidx_all)
        issue(0, 0)

        def body2(cc, carry):
            for sub in range(2):
                c = cc * 2 + sub
                issue(c + 1, 1 - sub)
                drain(sub)
                compute(sub)
            return carry

        assert nit % 2 == 1
        lax.fori_loop(0, nit // 2, body2, 0)
        drain(0)
        compute(0)

        pltpu.sync_copy(a_v, out_hbm.at[wid, 0])
        pltpu.sync_copy(b_v, out_hbm.at[wid, 1])

    return edge_kernel


def _finalize_body(a_ref, b_ref, out_ref):
    a = jnp.sum(a_ref[...], axis=0, keepdims=True)
    b = jnp.sum(b_ref[...], axis=0, keepdims=True)
    cut = a - b
    per = jnp.where(a > _EPS, cut / jnp.maximum(a, _EPS), 0.0)
    out_ref[...] = jnp.sum(per, axis=1, keepdims=True)


@jax.jit
def kernel(gat_refined_patch_features, patch_graph_edge_index,
           num_expected_segments, W_pred):
    x = gat_refined_patch_features
    ei = patch_graph_edge_index
    n, d = x.shape
    k = W_pred.shape[1]
    e_total = ei.shape[1]

    p = pl.pallas_call(
        _softmax_body,
        out_shape=jax.ShapeDtypeStruct((n, k), jnp.float32),
    )(x, W_pred)

    info = plsc.get_sparse_core_info()
    nc, ns = info.num_cores, info.num_subcores
    nw = nc * ns
    x2 = lax.bitcast_convert_type(
        x.astype(jnp.bfloat16).reshape(n, d // 2, 2), jnp.int32)
    edge_kernel = _make_edge_kernel(n, e_total, d, k, nc, ns, block=80)
    parts = edge_kernel(ei[0], ei[1], x2, p)

    a_parts = parts[:, 0].transpose(0, 2, 1).reshape(nw * 16, 16)
    b_parts = parts[:, 1].transpose(0, 2, 1).reshape(nw * 16, 16)
    out = pl.pallas_call(
        _finalize_body,
        out_shape=jax.ShapeDtypeStruct((1, 1), jnp.float32),
    )(a_parts, b_parts)
    l_partition = out[0, 0]
    return (l_partition, p)


# speedup vs baseline: 1.2389x; 1.2389x over previous
import functools

import jax
import jax.numpy as jnp
from jax import lax
from jax.experimental import pallas as pl
from jax.experimental.pallas import tpu as pltpu
from jax.experimental.pallas import tpu_sc as plsc

_EPS = 1e-08


def _softmax_body(x_ref, w_ref, p_ref):
    logits = jnp.dot(x_ref[...], w_ref[...], preferred_element_type=jnp.float32)
    m = jnp.max(logits, axis=1, keepdims=True)
    e = jnp.exp(logits - m)
    p_ref[...] = e / jnp.sum(e, axis=1, keepdims=True)


def _make_edge_kernel(n, e_total, d, k, nc, ns, block):
    nw = nc * ns
    epw = e_total // nw
    nit = epw // block
    assert e_total % nw == 0 and epw % block == 0
    assert block % 16 == 0 and block % 8 == 0 and block <= 128

    mesh = plsc.VectorSubcoreMesh(core_axis_name="c", subcore_axis_name="s")

    @functools.partial(
        pl.kernel,
        mesh=mesh,
        compiler_params=pltpu.CompilerParams(
            needs_layout_passes=False, use_tc_tiling_on_sc=False),
        out_type=jax.ShapeDtypeStruct((nw, 2, 16, 16), jnp.float32),
        scratch_types=[
            pltpu.VMEM((epw,), jnp.int32),
            pltpu.VMEM((epw,), jnp.int32),
            pltpu.VMEM((block, d), jnp.float32),
            pltpu.VMEM((block, d), jnp.float32),
            pltpu.VMEM((block, d), jnp.float32),
            pltpu.VMEM((block, d), jnp.float32),
            pltpu.VMEM((block, k), jnp.float32),
            pltpu.VMEM((block, k), jnp.float32),
            pltpu.VMEM((block, k), jnp.float32),
            pltpu.VMEM((block, k), jnp.float32),
            pltpu.VMEM((k, 16), jnp.float32),
            pltpu.VMEM((k, 16), jnp.float32),
            pltpu.SemaphoreType.DMA,
            pltpu.SemaphoreType.DMA,
        ],
    )
    def edge_kernel(src_hbm, tgt_hbm, x_hbm, p_hbm, out_hbm,
                    sidx_all, tidx_all, srow0, srow1, trow0, trow1,
                    sp0, sp1, tp0, tp1, a_v, b_v, sem0, sem1):
        wid = lax.axis_index("s") * nc + lax.axis_index("c")
        zero16 = jnp.zeros((16,), jnp.float32)
        for kk in range(k):
            a_v[kk, :] = zero16
            b_v[kk, :] = zero16

        lanes = lax.iota(jnp.int32, 16)
        bufs = ((srow0, trow0, sp0, tp0, sem0), (srow1, trow1, sp1, tp1, sem1))

        def issue(c, bi):
            srow, trow, sp, tp, sem = bufs[bi]
            si = sidx_all.at[pl.ds(c * block, block)]
            ti = tidx_all.at[pl.ds(c * block, block)]
            pltpu.async_copy(x_hbm.at[si], srow, sem)
            pltpu.async_copy(x_hbm.at[ti], trow, sem)
            pltpu.async_copy(p_hbm.at[si], sp, sem)
            pltpu.async_copy(p_hbm.at[ti], tp, sem)

        def drain(bi):
            srow, trow, sp, tp, sem = bufs[bi]
            pltpu.make_async_copy(x_hbm.at[pl.ds(0, block)], srow, sem).wait()
            pltpu.make_async_copy(x_hbm.at[pl.ds(0, block)], trow, sem).wait()
            pltpu.make_async_copy(p_hbm.at[pl.ds(0, block)], sp, sem).wait()
            pltpu.make_async_copy(p_hbm.at[pl.ds(0, block)], tp, sem).wait()

        swz = (lanes * 17) & (d - 1)

        def compute(bi):
            srow, trow, sp, tp, _ = bufs[bi]
            for g in range(block // 16):
                rows = lanes + (g * 16)

                @plsc.parallel_loop(0, d, 1, unroll=8,
                                    carry=jnp.zeros((16,), jnp.float32))
                def dist(j, acc):
                    col = (swz + j) & (d - 1)
                    s = plsc.load_gather(srow, [rows, col])
                    t = plsc.load_gather(trow, [rows, col])
                    diff = s - t
                    return acc + diff * diff
                w = jnp.exp(dist * -0.5)
                for kk in range(k):
                    col = (lanes + kk) & (k - 1)
                    ps = plsc.load_gather(sp, [rows, col])
                    pt = plsc.load_gather(tp, [rows, col])
                    wp = w * ps
                    plsc.addupdate_scatter(a_v, [col, lanes], wp)
                    plsc.addupdate_scatter(b_v, [col, lanes], wp * pt)

        pltpu.sync_copy(src_hbm.at[pl.ds(wid * epw, epw)], sidx_all)
        pltpu.sync_copy(tgt_hbm.at[pl.ds(wid * epw, epw)], tidx_all)
        issue(0, 0)

        def body2(cc, carry):
            for sub in range(2):
                c = cc * 2 + sub
                issue(c + 1, 1 - sub)
                drain(sub)
                compute(sub)
            return carry

        assert nit % 2 == 1
        lax.fori_loop(0, nit // 2, body2, 0)
        drain(0)
        compute(0)

        pltpu.sync_copy(a_v, out_hbm.at[wid, 0])
        pltpu.sync_copy(b_v, out_hbm.at[wid, 1])

    return edge_kernel


def _finalize_body(a_ref, b_ref, out_ref):
    a = jnp.sum(a_ref[...], axis=0, keepdims=True)
    b = jnp.sum(b_ref[...], axis=0, keepdims=True)
    cut = a - b
    per = jnp.where(a > _EPS, cut / jnp.maximum(a, _EPS), 0.0)
    out_ref[...] = jnp.sum(per, axis=1, keepdims=True)


@jax.jit
def kernel(gat_refined_patch_features, patch_graph_edge_index,
           num_expected_segments, W_pred):
    x = gat_refined_patch_features
    ei = patch_graph_edge_index
    n, d = x.shape
    k = W_pred.shape[1]
    e_total = ei.shape[1]

    p = pl.pallas_call(
        _softmax_body,
        out_shape=jax.ShapeDtypeStruct((n, k), jnp.float32),
    )(x, W_pred)

    info = plsc.get_sparse_core_info()
    nc, ns = info.num_cores, info.num_subcores
    nw = nc * ns
    edge_kernel = _make_edge_kernel(n, e_total, d, k, nc, ns, block=80)
    parts = edge_kernel(ei[0], ei[1], x, p)

    a_parts = parts[:, 0].transpose(0, 2, 1).reshape(nw * 16, 16)
    b_parts = parts[:, 1].transpose(0, 2, 1).reshape(nw * 16, 16)
    out = pl.pallas_call(
        _finalize_body,
        out_shape=jax.ShapeDtypeStruct((1, 1), jnp.float32),
    )(a_parts, b_parts)
    l_partition = out[0, 0]
    return (l_partition, p)
